# baseline (speedup 1.0000x reference)
import functools

import jax
import jax.numpy as jnp
from jax import lax
from jax.experimental import pallas as pl
from jax.experimental.pallas import tpu as pltpu
from jax.experimental.pallas import tpu_sc as plsc

NUM_MOVIES = 1000000
FEAT_DIM = 64
EMBED_DIM = 32
BATCH = 16384

NC = 2
NS = 16
NW = NC * NS
B_PER_W = BATCH // NW
CHUNK = 128
NCHUNK = B_PER_W // CHUNK


def _gather_body(idx_hbm, table_hbm, out_hbm, idx_v, rows_v, sem):
    wid = lax.axis_index("s") * NC + lax.axis_index("c")
    base = wid * B_PER_W
    pltpu.sync_copy(idx_hbm.at[wid], idx_v)
    copies = []
    for j in range(NCHUNK):
        copies.append(
            pltpu.async_copy(
                table_hbm.at[idx_v.at[j]],
                rows_v.at[pl.ds(j * CHUNK, CHUNK)],
                sem,
            )
        )
    for c in copies:
        c.wait()
    pltpu.sync_copy(rows_v, out_hbm.at[pl.ds(base, B_PER_W)])


@jax.jit
def _sc_gather(idx_3d, table):
    mesh = plsc.VectorSubcoreMesh(core_axis_name="c", subcore_axis_name="s")
    return pl.kernel(
        _gather_body,
        mesh=mesh,
        out_type=jax.ShapeDtypeStruct((BATCH, EMBED_DIM), jnp.float32),
        scratch_types=[
            pltpu.VMEM((NCHUNK, CHUNK), jnp.int32),
            pltpu.VMEM((B_PER_W, EMBED_DIM), jnp.float32),
            pltpu.SemaphoreType.DMA,
        ],
        compiler_params=pltpu.CompilerParams(use_tc_tiling_on_sc=False),
    )(idx_3d, table)


BLK = 2048


def _mlp_body(e_ref, f_ref, w1e_ref, w1f_ref, b1_ref, w2_ref, b2_ref,
              w3_ref, b3_ref, o_ref):
    h = jnp.dot(e_ref[...], w1e_ref[...], preferred_element_type=jnp.float32)
    h = h + jnp.dot(f_ref[...], w1f_ref[...],
                    preferred_element_type=jnp.float32)
    h = jnp.maximum(h + b1_ref[...], 0.0)
    h = jnp.maximum(
        jnp.dot(h, w2_ref[...], preferred_element_type=jnp.float32)
        + b2_ref[...], 0.0)
    o_ref[...] = (
        jnp.dot(h, w3_ref[...], preferred_element_type=jnp.float32)
        + b3_ref[...])


def _full(shape):
    return pl.BlockSpec(shape, lambda i: (0, 0))


@jax.jit
def _tc_mlp(embed, feat, w1e_t, w1f_t, b1, w2_t, b2, w3_t, b3):
    grid = (BATCH // BLK,)
    return pl.pallas_call(
        _mlp_body,
        grid=grid,
        in_specs=[
            pl.BlockSpec((BLK, EMBED_DIM), lambda i: (i, 0)),
            pl.BlockSpec((BLK, FEAT_DIM), lambda i: (i, 0)),
            _full(w1e_t.shape),
            _full(w1f_t.shape),
            _full(b1.shape),
            _full(w2_t.shape),
            _full(b2.shape),
            _full(w3_t.shape),
            _full(b3.shape),
        ],
        out_specs=pl.BlockSpec((BLK, EMBED_DIM), lambda i: (i, 0)),
        out_shape=jax.ShapeDtypeStruct((BATCH, EMBED_DIM), jnp.float32),
    )(embed, feat, w1e_t, w1f_t, b1, w2_t, b2, w3_t, b3)


def kernel(movie_id, movie_features, table, W1, b1, W2, b2, W3, b3):
    idx_3d = movie_id.reshape(NW, NCHUNK, CHUNK)
    embed = _sc_gather(idx_3d, table)
    w1e_t = W1[:, :EMBED_DIM].T
    w1f_t = W1[:, EMBED_DIM:].T
    return _tc_mlp(embed, movie_features, w1e_t, w1f_t, b1.reshape(1, -1),
                   W2.T, b2.reshape(1, -1), W3.T, b3.reshape(1, -1))


# speedup vs baseline: 2.2486x; 2.2486x over previous
import jax
import jax.numpy as jnp
from jax import lax
from jax.experimental import pallas as pl
from jax.experimental.pallas import tpu as pltpu
from jax.experimental.pallas import tpu_sc as plsc

NUM_MOVIES = 1000000
FEAT_DIM = 64
EMBED_DIM = 32
BATCH = 16384

NC = 2
NS = 16
NW = NC * NS
B_PER_W = BATCH // NW
GRP = 8
OUT_W = 128


def _gather_body(idx_hbm, tab3_hbm, out_hbm, idx_v, buf_a, buf_b, out_v,
                 sem_a, sem_b):
    wid = lax.axis_index("s") * NC + lax.axis_index("c")
    base = wid * B_PER_W
    pltpu.sync_copy(idx_hbm.at[pl.ds(base, B_PER_W)], idx_v)

    def fire(buf, sem, v16, lane0):
        for j in range(GRP):
            maj = lax.shift_right_logical(v16[lane0 + j], 3)
            pltpu.async_copy(tab3_hbm.at[pl.ds(maj, 1)],
                             buf.at[pl.ds(j, 1)], sem)

    def drain(buf, sem):
        for j in range(GRP):
            pltpu.make_async_copy(tab3_hbm.at[pl.ds(0, 1)],
                                  buf.at[pl.ds(j, 1)], sem).wait()

    def select(buf, v16, lane0, r0):
        for j in range(GRP):
            sub = lax.bitwise_and(v16[lane0 + j], jnp.int32(7))
            out_v[r0 + j, pl.ds(0, 16)] = buf[j, sub, pl.ds(0, 16)]
            out_v[r0 + j, pl.ds(16, 16)] = buf[j, sub, pl.ds(16, 16)]

    v16_0 = idx_v[pl.ds(0, 16)]
    fire(buf_a, sem_a, v16_0, 0)
    fire(buf_b, sem_b, v16_0, GRP)
    n_iter = B_PER_W // (2 * GRP)

    def body(t, _):
        r0 = t * (2 * GRP)
        v16 = idx_v[pl.ds(r0, 16)]
        drain(buf_a, sem_a)
        select(buf_a, v16, 0, r0)

        @pl.when(t < n_iter - 1)
        def _():
            nv16 = idx_v[pl.ds(r0 + 16, 16)]
            fire(buf_a, sem_a, nv16, 0)

        drain(buf_b, sem_b)
        select(buf_b, v16, GRP, r0 + GRP)

        @pl.when(t < n_iter - 1)
        def _():
            nv16 = idx_v[pl.ds(r0 + 16, 16)]
            fire(buf_b, sem_b, nv16, GRP)

        return 0

    lax.fori_loop(0, n_iter, body, 0)
    pltpu.sync_copy(out_v, out_hbm.at[pl.ds(base, B_PER_W)])


@jax.jit
def _sc_gather(movie_id, tab3):
    mesh = plsc.VectorSubcoreMesh(core_axis_name="c", subcore_axis_name="s")
    return pl.kernel(
        _gather_body,
        mesh=mesh,
        out_type=jax.ShapeDtypeStruct((BATCH, OUT_W), jnp.float32),
        scratch_types=[
            pltpu.VMEM((B_PER_W,), jnp.int32),
            pltpu.VMEM((GRP, 8, EMBED_DIM), jnp.float32),
            pltpu.VMEM((GRP, 8, EMBED_DIM), jnp.float32),
            pltpu.VMEM((B_PER_W, OUT_W), jnp.float32),
            pltpu.SemaphoreType.DMA,
            pltpu.SemaphoreType.DMA,
        ],
        compiler_params=pltpu.CompilerParams(use_tc_tiling_on_sc=True),
    )(movie_id, tab3)


BLK = 2048


def _mlp_body(e_ref, f_ref, w1e_ref, w1f_ref, b1_ref, w2_ref, b2_ref,
              w3_ref, b3_ref, o_ref):
    e = e_ref[...][:, :EMBED_DIM]
    h = jnp.dot(e, w1e_ref[...], preferred_element_type=jnp.float32)
    h = h + jnp.dot(f_ref[...], w1f_ref[...],
                    preferred_element_type=jnp.float32)
    h = jnp.maximum(h + b1_ref[...], 0.0)
    h = jnp.maximum(
        jnp.dot(h, w2_ref[...], preferred_element_type=jnp.float32)
        + b2_ref[...], 0.0)
    o_ref[...] = (
        jnp.dot(h, w3_ref[...], preferred_element_type=jnp.float32)
        + b3_ref[...])


def _full(shape):
    return pl.BlockSpec(shape, lambda i: (0, 0))


@jax.jit
def _tc_mlp(embed, feat, w1e_t, w1f_t, b1, w2_t, b2, w3_t, b3):
    grid = (BATCH // BLK,)
    return pl.pallas_call(
        _mlp_body,
        grid=grid,
        in_specs=[
            pl.BlockSpec((BLK, OUT_W), lambda i: (i, 0)),
            pl.BlockSpec((BLK, FEAT_DIM), lambda i: (i, 0)),
            _full(w1e_t.shape),
            _full(w1f_t.shape),
            _full(b1.shape),
            _full(w2_t.shape),
            _full(b2.shape),
            _full(w3_t.shape),
            _full(b3.shape),
        ],
        out_specs=pl.BlockSpec((BLK, EMBED_DIM), lambda i: (i, 0)),
        out_shape=jax.ShapeDtypeStruct((BATCH, EMBED_DIM), jnp.float32),
    )(embed, feat, w1e_t, w1f_t, b1, w2_t, b2, w3_t, b3)


def kernel(movie_id, movie_features, table, W1, b1, W2, b2, W3, b3):
    tab3 = table.reshape(NUM_MOVIES // 8, 8, EMBED_DIM)
    embed = _sc_gather(movie_id, tab3)
    w1e_t = W1[:, :EMBED_DIM].T
    w1f_t = W1[:, EMBED_DIM:].T
    return _tc_mlp(embed, movie_features, w1e_t, w1f_t, b1.reshape(1, -1),
                   W2.T, b2.reshape(1, -1), W3.T, b3.reshape(1, -1))
